# baseline (speedup 1.0000x reference)
import jax
import jax.numpy as jnp
from jax import lax
from jax.experimental import pallas as pl
from jax.experimental.pallas import tpu as pltpu
from jax.experimental.pallas import tpu_sc as plsc

N = 262144
H = 512
K = N // 4
T = 16
C = N // T
NV = C // 16
HNV = NV // 2
HC = C // 2
BINS = 1024
GRID = BINS * T
SLICE = GRID // T
KPT = K // T
HCOPIES = 8


def _iota():
    return lax.iota(jnp.int32, 16)


def _sc_body(ego_hbm, nb_hbm, rows_hbm, cols_hbm,
             key_loc, nb_loc, idx_loc, pos0_loc, pos1_loc,
             hist_loc, off_loc, gidx_loc, sidx_loc, slc_loc, tot_loc,
             akey, aidx, grid_sh, tot_sh,
             sem0, sem1, sem2, sem3):
    cid = lax.axis_index("c")
    sid = lax.axis_index("s")

    @pl.when(cid == 0)
    def _():
        wid = sid
        base = wid * C
        zeros16 = jnp.zeros((16,), jnp.int32)
        ones16 = jnp.ones((16,), jnp.int32)

        pltpu.sync_copy(ego_hbm.at[pl.ds(base, C)], key_loc)
        pltpu.sync_copy(nb_hbm.at[pl.ds(base, C)], nb_loc)

        @plsc.parallel_loop(0, NV, unroll=4)
        def _(j):
            e = key_loc[pl.ds(j * 16, 16)]
            nbv = nb_loc[pl.ds(j * 16, 16)]
            key_loc[pl.ds(j * 16, 16)] = (1.0 - e) * nbv
            idx_loc[pl.ds(j * 16, 16)] = base + j * 16 + _iota()

        @plsc.parallel_loop(0, BINS // 16, unroll=4)
        def _(j):
            gidx_loc[pl.ds(j * 16, 16)] = (j * 16 + _iota()) * T + wid
            sidx_loc[pl.ds(j * 16, 16)] = wid * SLICE + j * 16 + _iota()

        for p, shift in enumerate((0, 10, 20)):
            cpidx = None
            if p == 1:
                cpk0 = pltpu.async_copy(akey.at[pl.ds(base, HC)],
                                        key_loc.at[pl.ds(0, HC)], sem0)
                cpk1 = pltpu.async_copy(akey.at[pl.ds(base + HC, HC)],
                                        key_loc.at[pl.ds(HC, HC)], sem2)
                cpidx = pltpu.async_copy(aidx.at[pl.ds(base, C)], idx_loc,
                                         sem1)
                cpk0.wait()
            elif p == 2:
                cpk0 = pltpu.async_copy(aidx.at[pl.ds(base, HC)],
                                        idx_loc.at[pl.ds(0, HC)], sem0)
                cpk1 = pltpu.async_copy(aidx.at[pl.ds(base + HC, HC)],
                                        idx_loc.at[pl.ds(HC, HC)], sem2)
                cpk0.wait()

            @plsc.parallel_loop(0, HCOPIES * BINS // 16, unroll=8)
            def _(j):
                hist_loc[pl.ds(j * 16, 16)] = zeros16

            def _histrank(j):
                if p < 2:
                    k32 = plsc.bitcast(key_loc[pl.ds(j * 16, 16)], jnp.int32)
                    d = 1023 - ((k32 >> shift) & 1023)
                else:
                    d = lax.shift_right_logical(idx_loc[pl.ds(j * 16, 16)],
                                                18)
                cnt, _unused = plsc.scan_count(d)
                nb_loc[pl.ds(j * 16, 16)] = plsc.bitcast(cnt, jnp.float32)
                plsc.addupdate_scatter(hist_loc, [d + (j % HCOPIES) * BINS],
                                       ones16)

            plsc.parallel_loop(0, HNV, unroll=HCOPIES)(_histrank)
            if p > 0:
                cpk1.wait()
            plsc.parallel_loop(HNV, NV, unroll=HCOPIES)(_histrank)

            @plsc.parallel_loop(0, BINS // 16, unroll=4)
            def _(j):
                acc = hist_loc[pl.ds(j * 16, 16)]
                for c in range(1, HCOPIES):
                    acc = acc + hist_loc[pl.ds(c * BINS + j * 16, 16)]
                hist_loc[pl.ds(j * 16, 16)] = acc

            pltpu.async_copy(hist_loc.at[pl.ds(0, BINS)],
                             grid_sh.at[gidx_loc], sem2).wait()
            plsc.subcore_barrier()

            pltpu.async_copy(grid_sh.at[sidx_loc], slc_loc, sem2).wait()

            fifteen = jnp.full((16,), 15, jnp.int32)

            def p_scan(j, carry):
                v = slc_loc[pl.ds(j * 16, 16)]
                inc = plsc.cumsum(v)
                slc_loc[pl.ds(j * 16, 16)] = inc - v + carry
                return carry + lax.gather(
                    inc, fifteen[:, None],
                    lax.GatherDimensionNumbers(
                        offset_dims=(), collapsed_slice_dims=(0,),
                        start_index_map=(0,)),
                    (1,), mode=lax.GatherScatterMode.PROMISE_IN_BOUNDS)

            totv = lax.fori_loop(0, SLICE // 16, p_scan,
                                 jnp.zeros((16,), jnp.int32), unroll=2)
            tot_loc[pl.ds(0, 16)] = jnp.where(_iota() == 0, totv, 0)
            ti = jnp.where(_iota() == 0, wid, T + wid)
            cps = pltpu.async_copy(slc_loc, grid_sh.at[sidx_loc], sem2)
            cpt = pltpu.async_copy(tot_loc.at[pl.ds(0, 16)], tot_sh.at[ti], sem3)
            cps.wait()
            cpt.wait()
            plsc.subcore_barrier()

            pltpu.sync_copy(tot_sh, tot_loc)
            cpo = pltpu.async_copy(grid_sh.at[gidx_loc], off_loc, sem2)
            t16 = tot_loc[pl.ds(0, 16)]
            ctot = plsc.cumsum(t16) - t16
            tot_loc[pl.ds(0, 16)] = ctot
            cpo.wait()

            @plsc.parallel_loop(0, BINS // 16, unroll=4)
            def _(j):
                gi = (j * 16 + _iota()) * T + wid
                carry = plsc.load_gather(tot_loc, [gi >> 10])
                off_loc[pl.ds(j * 16, 16)] = off_loc[pl.ds(j * 16, 16)] + carry

            def make_perm(pref, joff):
                def p_perm(j, _):
                    if p < 2:
                        k32 = plsc.bitcast(key_loc[pl.ds(joff + j * 16, 16)],
                                           jnp.int32)
                        d = 1023 - ((k32 >> shift) & 1023)
                    else:
                        d = lax.shift_right_logical(
                            idx_loc[pl.ds(joff + j * 16, 16)], 18)
                    off = plsc.load_gather(off_loc, [d])
                    cnt = plsc.bitcast(nb_loc[pl.ds(joff + j * 16, 16)],
                                       jnp.int32)
                    pref[pl.ds(j * 16, 16)] = off + cnt - 1
                    plsc.addupdate_scatter(off_loc, [d], ones16)
                    return 0
                return p_perm

            if cpidx is not None:
                cpidx.wait()
            if p == 1:
                @plsc.parallel_loop(0, NV, unroll=4)
                def _(j):
                    k32 = plsc.bitcast(key_loc[pl.ds(j * 16, 16)], jnp.int32)
                    df3 = 1023 - ((k32 >> 20) & 1023)
                    idx_loc[pl.ds(j * 16, 16)] = \
                        (df3 << 18) | idx_loc[pl.ds(j * 16, 16)]

            lax.fori_loop(0, HNV, make_perm(pos0_loc, 0), 0, unroll=4)
            if p == 0:
                cpk0 = pltpu.async_copy(key_loc.at[pl.ds(0, HC)],
                                        akey.at[pos0_loc], sem0)
            cpi0 = pltpu.async_copy(idx_loc.at[pl.ds(0, HC)],
                                    aidx.at[pos0_loc], sem1)
            lax.fori_loop(0, HNV, make_perm(pos1_loc, HC), 0, unroll=4)
            if p == 0:
                cpk1 = pltpu.async_copy(key_loc.at[pl.ds(HC, HC)],
                                        akey.at[pos1_loc], sem2)
            cpi1 = pltpu.async_copy(idx_loc.at[pl.ds(HC, HC)],
                                    aidx.at[pos1_loc], sem3)
            if p == 0:
                cpk0.wait()
                cpk1.wait()
            cpi0.wait()
            cpi1.wait()
            plsc.subcore_barrier()

        obase = wid * KPT
        pltpu.sync_copy(aidx.at[pl.ds(obase, KPT)], idx_loc.at[pl.ds(0, KPT)])

        @plsc.parallel_loop(0, KPT // 16, unroll=4)
        def _(j):
            v = idx_loc[pl.ds(j * 16, 16)] & 0x3FFFF
            pos0_loc[pl.ds(j * 16, 16)] = lax.shift_right_logical(v, 9)
            pos1_loc[pl.ds(j * 16, 16)] = v & (H - 1)

        pltpu.sync_copy(pos0_loc.at[pl.ds(0, KPT)],
                        rows_hbm.at[pl.ds(obase, KPT)])
        pltpu.sync_copy(pos1_loc.at[pl.ds(0, KPT)],
                        cols_hbm.at[pl.ds(obase, KPT)])


@jax.jit
def _run(ego_flat, nb_flat):
    mesh = plsc.VectorSubcoreMesh(core_axis_name="c", subcore_axis_name="s")
    f = pl.kernel(
        _sc_body,
        out_type=(jax.ShapeDtypeStruct((K,), jnp.int32),
                  jax.ShapeDtypeStruct((K,), jnp.int32)),
        mesh=mesh,
        compiler_params=pltpu.CompilerParams(needs_layout_passes=False),
        scratch_types=[
            pltpu.VMEM((C,), jnp.float32),
            pltpu.VMEM((C,), jnp.float32),
            pltpu.VMEM((C,), jnp.int32),
            pltpu.VMEM((HC,), jnp.int32),
            pltpu.VMEM((HC,), jnp.int32),
            pltpu.VMEM((HCOPIES * BINS,), jnp.int32),
            pltpu.VMEM((BINS,), jnp.int32),
            pltpu.VMEM((BINS,), jnp.int32),
            pltpu.VMEM((SLICE,), jnp.int32),
            pltpu.VMEM((SLICE,), jnp.int32),
            pltpu.VMEM((2 * T,), jnp.int32),
            pltpu.VMEM_SHARED((N,), jnp.float32),
            pltpu.VMEM_SHARED((N,), jnp.int32),
            pltpu.VMEM_SHARED((GRID,), jnp.int32),
            pltpu.VMEM_SHARED((2 * T,), jnp.int32),
            pltpu.SemaphoreType.DMA,
            pltpu.SemaphoreType.DMA,
            pltpu.SemaphoreType.DMA,
            pltpu.SemaphoreType.DMA,
        ],
    )
    return f(ego_flat, nb_flat)


def kernel(ego_conf, nb_conf, delta=0.25):
    del delta
    rows_idx, cols_idx = _run(ego_conf.reshape(-1), nb_conf.reshape(-1))
    return (rows_idx, cols_idx)


# speedup vs baseline: 1.0009x; 1.0009x over previous
import jax
import jax.numpy as jnp
from jax import lax
from jax.experimental import pallas as pl
from jax.experimental.pallas import tpu as pltpu
from jax.experimental.pallas import tpu_sc as plsc

N = 262144
H = 512
K = N // 4
T = 16
C = N // T
NV = C // 16
HNV = NV // 2
HC = C // 2
BINS = 1024
GRID = BINS * T
SLICE = GRID // T
KPT = K // T
HCOPIES = 8


def _iota():
    return lax.iota(jnp.int32, 16)


def _sc_body(ego_hbm, nb_hbm, rows_hbm, cols_hbm,
             key_loc, nb_loc, dig_loc, idx_loc, pos0_loc, pos1_loc,
             hist_loc, off_loc, gidx_loc, sidx_loc, slc_loc, tot_loc,
             akey, aidx, grid_sh, tot_sh,
             sem0, sem1, sem2, sem3):
    cid = lax.axis_index("c")
    sid = lax.axis_index("s")

    @pl.when(cid == 0)
    def _():
        wid = sid
        base = wid * C
        zeros16 = jnp.zeros((16,), jnp.int32)
        ones16 = jnp.ones((16,), jnp.int32)

        pltpu.sync_copy(ego_hbm.at[pl.ds(base, C)], key_loc)
        pltpu.sync_copy(nb_hbm.at[pl.ds(base, C)], nb_loc)

        @plsc.parallel_loop(0, NV, unroll=4)
        def _(j):
            e = key_loc[pl.ds(j * 16, 16)]
            nbv = nb_loc[pl.ds(j * 16, 16)]
            key_loc[pl.ds(j * 16, 16)] = (1.0 - e) * nbv
            idx_loc[pl.ds(j * 16, 16)] = base + j * 16 + _iota()

        @plsc.parallel_loop(0, BINS // 16, unroll=4)
        def _(j):
            gidx_loc[pl.ds(j * 16, 16)] = (j * 16 + _iota()) * T + wid
            sidx_loc[pl.ds(j * 16, 16)] = wid * SLICE + j * 16 + _iota()

        for p, shift in enumerate((0, 10, 20)):
            cpidx = None
            if p == 1:
                cpk0 = pltpu.async_copy(akey.at[pl.ds(base, HC)],
                                        key_loc.at[pl.ds(0, HC)], sem0)
                cpk1 = pltpu.async_copy(akey.at[pl.ds(base + HC, HC)],
                                        key_loc.at[pl.ds(HC, HC)], sem2)
                cpidx = pltpu.async_copy(aidx.at[pl.ds(base, C)], idx_loc,
                                         sem1)
                cpk0.wait()
            elif p == 2:
                cpk0 = pltpu.async_copy(aidx.at[pl.ds(base, HC)],
                                        idx_loc.at[pl.ds(0, HC)], sem0)
                cpk1 = pltpu.async_copy(aidx.at[pl.ds(base + HC, HC)],
                                        idx_loc.at[pl.ds(HC, HC)], sem2)
                cpk0.wait()

            @plsc.parallel_loop(0, HCOPIES * BINS // 16, unroll=8)
            def _(j):
                hist_loc[pl.ds(j * 16, 16)] = zeros16

            def _histrank(j):
                if p < 2:
                    k32 = plsc.bitcast(key_loc[pl.ds(j * 16, 16)], jnp.int32)
                    d = 1023 - ((k32 >> shift) & 1023)
                else:
                    d = lax.shift_right_logical(idx_loc[pl.ds(j * 16, 16)],
                                                18)
                cnt, _unused = plsc.scan_count(d)
                nb_loc[pl.ds(j * 16, 16)] = plsc.bitcast(cnt, jnp.float32)
                dig_loc[pl.ds(j * 16, 16)] = plsc.bitcast(d, jnp.float32)
                plsc.addupdate_scatter(hist_loc, [d + (j % HCOPIES) * BINS],
                                       ones16)

            plsc.parallel_loop(0, HNV, unroll=HCOPIES)(_histrank)
            if p > 0:
                cpk1.wait()
            plsc.parallel_loop(HNV, NV, unroll=HCOPIES)(_histrank)

            @plsc.parallel_loop(0, BINS // 16, unroll=4)
            def _(j):
                acc = hist_loc[pl.ds(j * 16, 16)]
                for c in range(1, HCOPIES):
                    acc = acc + hist_loc[pl.ds(c * BINS + j * 16, 16)]
                hist_loc[pl.ds(j * 16, 16)] = acc

            pltpu.async_copy(hist_loc.at[pl.ds(0, BINS)],
                             grid_sh.at[gidx_loc], sem2).wait()
            plsc.subcore_barrier()

            pltpu.async_copy(grid_sh.at[sidx_loc], slc_loc, sem2).wait()

            def p_scan(j, carry):
                v = slc_loc[pl.ds(j * 16, 16)]
                inc = plsc.cumsum(v)
                slc_loc[pl.ds(j * 16, 16)] = inc - v + carry
                return carry + jnp.sum(v)

            total = lax.fori_loop(0, SLICE // 16, p_scan, jnp.int32(0))
            tot_loc[pl.ds(0, 16)] = jnp.where(_iota() == 0, total, 0)
            ti = jnp.where(_iota() == 0, wid, T + wid)
            cps = pltpu.async_copy(slc_loc, grid_sh.at[sidx_loc], sem2)
            cpt = pltpu.async_copy(tot_loc.at[pl.ds(0, 16)], tot_sh.at[ti], sem3)
            cps.wait()
            cpt.wait()
            plsc.subcore_barrier()

            pltpu.sync_copy(tot_sh, tot_loc)
            cpo = pltpu.async_copy(grid_sh.at[gidx_loc], off_loc, sem2)
            t16 = tot_loc[pl.ds(0, 16)]
            ctot = plsc.cumsum(t16) - t16
            tot_loc[pl.ds(0, 16)] = ctot
            cpo.wait()

            @plsc.parallel_loop(0, BINS // 16, unroll=4)
            def _(j):
                gi = (j * 16 + _iota()) * T + wid
                carry = plsc.load_gather(tot_loc, [gi >> 10])
                off_loc[pl.ds(j * 16, 16)] = off_loc[pl.ds(j * 16, 16)] + carry

            def make_perm(pref, joff):
                def p_perm(j, _):
                    d = plsc.bitcast(dig_loc[pl.ds(joff + j * 16, 16)],
                                     jnp.int32)
                    off = plsc.load_gather(off_loc, [d])
                    cnt = plsc.bitcast(nb_loc[pl.ds(joff + j * 16, 16)],
                                       jnp.int32)
                    pref[pl.ds(j * 16, 16)] = off + cnt - 1
                    plsc.addupdate_scatter(off_loc, [d], ones16)
                    return 0
                return p_perm

            if cpidx is not None:
                cpidx.wait()
            if p == 1:
                @plsc.parallel_loop(0, NV, unroll=4)
                def _(j):
                    k32 = plsc.bitcast(key_loc[pl.ds(j * 16, 16)], jnp.int32)
                    df3 = 1023 - ((k32 >> 20) & 1023)
                    idx_loc[pl.ds(j * 16, 16)] = \
                        (df3 << 18) | idx_loc[pl.ds(j * 16, 16)]

            lax.fori_loop(0, HNV, make_perm(pos0_loc, 0), 0)
            if p == 0:
                cpk0 = pltpu.async_copy(key_loc.at[pl.ds(0, HC)],
                                        akey.at[pos0_loc], sem0)
            cpi0 = pltpu.async_copy(idx_loc.at[pl.ds(0, HC)],
                                    aidx.at[pos0_loc], sem1)
            lax.fori_loop(0, HNV, make_perm(pos1_loc, HC), 0)
            if p == 0:
                cpk1 = pltpu.async_copy(key_loc.at[pl.ds(HC, HC)],
                                        akey.at[pos1_loc], sem2)
            cpi1 = pltpu.async_copy(idx_loc.at[pl.ds(HC, HC)],
                                    aidx.at[pos1_loc], sem3)
            if p == 0:
                cpk0.wait()
                cpk1.wait()
            cpi0.wait()
            cpi1.wait()
            plsc.subcore_barrier()

        obase = wid * KPT
        pltpu.sync_copy(aidx.at[pl.ds(obase, KPT)], idx_loc.at[pl.ds(0, KPT)])

        @plsc.parallel_loop(0, KPT // 16, unroll=4)
        def _(j):
            v = idx_loc[pl.ds(j * 16, 16)] & 0x3FFFF
            pos0_loc[pl.ds(j * 16, 16)] = lax.shift_right_logical(v, 9)
            pos1_loc[pl.ds(j * 16, 16)] = v & (H - 1)

        pltpu.sync_copy(pos0_loc.at[pl.ds(0, KPT)],
                        rows_hbm.at[pl.ds(obase, KPT)])
        pltpu.sync_copy(pos1_loc.at[pl.ds(0, KPT)],
                        cols_hbm.at[pl.ds(obase, KPT)])


@jax.jit
def _run(ego_flat, nb_flat):
    mesh = plsc.VectorSubcoreMesh(core_axis_name="c", subcore_axis_name="s")
    f = pl.kernel(
        _sc_body,
        out_type=(jax.ShapeDtypeStruct((K,), jnp.int32),
                  jax.ShapeDtypeStruct((K,), jnp.int32)),
        mesh=mesh,
        compiler_params=pltpu.CompilerParams(needs_layout_passes=False),
        scratch_types=[
            pltpu.VMEM((C,), jnp.float32),
            pltpu.VMEM((C,), jnp.float32),
            pltpu.VMEM((C,), jnp.float32),
            pltpu.VMEM((C,), jnp.int32),
            pltpu.VMEM((HC,), jnp.int32),
            pltpu.VMEM((HC,), jnp.int32),
            pltpu.VMEM((HCOPIES * BINS,), jnp.int32),
            pltpu.VMEM((BINS,), jnp.int32),
            pltpu.VMEM((BINS,), jnp.int32),
            pltpu.VMEM((SLICE,), jnp.int32),
            pltpu.VMEM((SLICE,), jnp.int32),
            pltpu.VMEM((2 * T,), jnp.int32),
            pltpu.VMEM_SHARED((N,), jnp.float32),
            pltpu.VMEM_SHARED((N,), jnp.int32),
            pltpu.VMEM_SHARED((GRID,), jnp.int32),
            pltpu.VMEM_SHARED((2 * T,), jnp.int32),
            pltpu.SemaphoreType.DMA,
            pltpu.SemaphoreType.DMA,
            pltpu.SemaphoreType.DMA,
            pltpu.SemaphoreType.DMA,
        ],
    )
    return f(ego_flat, nb_flat)


def kernel(ego_conf, nb_conf, delta=0.25):
    del delta
    rows_idx, cols_idx = _run(ego_conf.reshape(-1), nb_conf.reshape(-1))
    return (rows_idx, cols_idx)


# speedup vs baseline: 1.0073x; 1.0064x over previous
import jax
import jax.numpy as jnp
from jax import lax
from jax.experimental import pallas as pl
from jax.experimental.pallas import tpu as pltpu
from jax.experimental.pallas import tpu_sc as plsc

N = 262144
H = 512
K = N // 4
T = 16
C = N // T
NV = C // 16
HNV = NV // 2
HC = C // 2
BINS = 1024
GRID = BINS * T
SLICE = GRID // T
KPT = K // T
HCOPIES = 8


def _iota():
    return lax.iota(jnp.int32, 16)


def _sc_body(ego_hbm, nb_hbm, rows_hbm, cols_hbm,
             key_loc, nb_loc, dig_loc, idx_loc, pos0_loc, pos1_loc,
             hist_loc, off_loc, gidx_loc, sidx_loc, slc_loc, tot_loc,
             akey, aidx, grid_sh, tot_sh,
             sem0, sem1, sem2, sem3):
    cid = lax.axis_index("c")
    sid = lax.axis_index("s")

    @pl.when(cid == 0)
    def _():
        wid = sid
        base = wid * C
        zeros16 = jnp.zeros((16,), jnp.int32)
        ones16 = jnp.ones((16,), jnp.int32)

        pltpu.sync_copy(ego_hbm.at[pl.ds(base, C)], key_loc)
        pltpu.sync_copy(nb_hbm.at[pl.ds(base, C)], nb_loc)

        @plsc.parallel_loop(0, NV, unroll=4)
        def _(j):
            e = key_loc[pl.ds(j * 16, 16)]
            nbv = nb_loc[pl.ds(j * 16, 16)]
            key_loc[pl.ds(j * 16, 16)] = (1.0 - e) * nbv
            idx_loc[pl.ds(j * 16, 16)] = base + j * 16 + _iota()

        @plsc.parallel_loop(0, BINS // 16, unroll=4)
        def _(j):
            gidx_loc[pl.ds(j * 16, 16)] = (j * 16 + _iota()) * T + wid
            sidx_loc[pl.ds(j * 16, 16)] = wid * SLICE + j * 16 + _iota()

        for p, shift in enumerate((0, 10, 20)):
            cpidx = None
            if p == 1:
                cpk0 = pltpu.async_copy(akey.at[pl.ds(base, HC)],
                                        key_loc.at[pl.ds(0, HC)], sem0)
                cpk1 = pltpu.async_copy(akey.at[pl.ds(base + HC, HC)],
                                        key_loc.at[pl.ds(HC, HC)], sem2)
                cpidx = pltpu.async_copy(aidx.at[pl.ds(base, C)], idx_loc,
                                         sem1)
            elif p == 2:
                cpk0 = pltpu.async_copy(aidx.at[pl.ds(base, HC)],
                                        idx_loc.at[pl.ds(0, HC)], sem0)
                cpk1 = pltpu.async_copy(aidx.at[pl.ds(base + HC, HC)],
                                        idx_loc.at[pl.ds(HC, HC)], sem2)

            @plsc.parallel_loop(0, HCOPIES * BINS // 16, unroll=8)
            def _(j):
                hist_loc[pl.ds(j * 16, 16)] = zeros16

            if p > 0:
                cpk0.wait()

            def _histrank(j):
                if p < 2:
                    k32 = plsc.bitcast(key_loc[pl.ds(j * 16, 16)], jnp.int32)
                    d = 1023 - ((k32 >> shift) & 1023)
                else:
                    d = lax.shift_right_logical(idx_loc[pl.ds(j * 16, 16)],
                                                18)
                cnt, _unused = plsc.scan_count(d)
                nb_loc[pl.ds(j * 16, 16)] = plsc.bitcast(cnt, jnp.float32)
                dig_loc[pl.ds(j * 16, 16)] = plsc.bitcast(d, jnp.float32)
                plsc.addupdate_scatter(hist_loc, [d + (j % HCOPIES) * BINS],
                                       ones16)

            plsc.parallel_loop(0, HNV, unroll=HCOPIES)(_histrank)
            if p > 0:
                cpk1.wait()
            plsc.parallel_loop(HNV, NV, unroll=HCOPIES)(_histrank)

            @plsc.parallel_loop(0, BINS // 16, unroll=4)
            def _(j):
                acc = hist_loc[pl.ds(j * 16, 16)]
                for c in range(1, HCOPIES):
                    acc = acc + hist_loc[pl.ds(c * BINS + j * 16, 16)]
                hist_loc[pl.ds(j * 16, 16)] = acc

            pltpu.async_copy(hist_loc.at[pl.ds(0, BINS)],
                             grid_sh.at[gidx_loc], sem2).wait()
            plsc.subcore_barrier()

            pltpu.async_copy(grid_sh.at[sidx_loc], slc_loc, sem2).wait()

            def p_scan(j, carry):
                v = slc_loc[pl.ds(j * 16, 16)]
                inc = plsc.cumsum(v)
                slc_loc[pl.ds(j * 16, 16)] = inc - v + carry
                return carry + jnp.sum(v)

            total = lax.fori_loop(0, SLICE // 16, p_scan, jnp.int32(0))
            tot_loc[pl.ds(0, 16)] = jnp.where(_iota() == 0, total, 0)
            ti = jnp.where(_iota() == 0, wid, T + wid)
            cps = pltpu.async_copy(slc_loc, grid_sh.at[sidx_loc], sem2)
            cpt = pltpu.async_copy(tot_loc.at[pl.ds(0, 16)], tot_sh.at[ti], sem3)
            cps.wait()
            cpt.wait()
            plsc.subcore_barrier()

            pltpu.sync_copy(tot_sh, tot_loc)
            cpo = pltpu.async_copy(grid_sh.at[gidx_loc], off_loc, sem2)
            t16 = tot_loc[pl.ds(0, 16)]
            ctot = plsc.cumsum(t16) - t16
            tot_loc[pl.ds(0, 16)] = ctot
            cpo.wait()

            @plsc.parallel_loop(0, BINS // 16, unroll=4)
            def _(j):
                gi = (j * 16 + _iota()) * T + wid
                carry = plsc.load_gather(tot_loc, [gi >> 10])
                off_loc[pl.ds(j * 16, 16)] = off_loc[pl.ds(j * 16, 16)] + carry

            def make_perm(pref, joff):
                def p_perm(j, _):
                    d = plsc.bitcast(dig_loc[pl.ds(joff + j * 16, 16)],
                                     jnp.int32)
                    off = plsc.load_gather(off_loc, [d])
                    cnt = plsc.bitcast(nb_loc[pl.ds(joff + j * 16, 16)],
                                       jnp.int32)
                    pref[pl.ds(j * 16, 16)] = off + cnt - 1
                    plsc.addupdate_scatter(off_loc, [d], ones16)
                    return 0
                return p_perm

            if cpidx is not None:
                cpidx.wait()
            if p == 1:
                @plsc.parallel_loop(0, NV, unroll=4)
                def _(j):
                    k32 = plsc.bitcast(key_loc[pl.ds(j * 16, 16)], jnp.int32)
                    df3 = 1023 - ((k32 >> 20) & 1023)
                    idx_loc[pl.ds(j * 16, 16)] = \
                        (df3 << 18) | idx_loc[pl.ds(j * 16, 16)]

            lax.fori_loop(0, HNV, make_perm(pos0_loc, 0), 0)
            if p == 0:
                cpk0 = pltpu.async_copy(key_loc.at[pl.ds(0, HC)],
                                        akey.at[pos0_loc], sem0)
            cpi0 = pltpu.async_copy(idx_loc.at[pl.ds(0, HC)],
                                    aidx.at[pos0_loc], sem1)
            lax.fori_loop(0, HNV, make_perm(pos1_loc, HC), 0)
            if p == 0:
                cpk1 = pltpu.async_copy(key_loc.at[pl.ds(HC, HC)],
                                        akey.at[pos1_loc], sem2)
            cpi1 = pltpu.async_copy(idx_loc.at[pl.ds(HC, HC)],
                                    aidx.at[pos1_loc], sem3)
            if p == 0:
                cpk0.wait()
                cpk1.wait()
            cpi0.wait()
            cpi1.wait()
            plsc.subcore_barrier()

        obase = wid * KPT
        pltpu.sync_copy(aidx.at[pl.ds(obase, KPT)], idx_loc.at[pl.ds(0, KPT)])

        @plsc.parallel_loop(0, KPT // 16, unroll=4)
        def _(j):
            v = idx_loc[pl.ds(j * 16, 16)] & 0x3FFFF
            pos0_loc[pl.ds(j * 16, 16)] = lax.shift_right_logical(v, 9)
            pos1_loc[pl.ds(j * 16, 16)] = v & (H - 1)

        pltpu.sync_copy(pos0_loc.at[pl.ds(0, KPT)],
                        rows_hbm.at[pl.ds(obase, KPT)])
        pltpu.sync_copy(pos1_loc.at[pl.ds(0, KPT)],
                        cols_hbm.at[pl.ds(obase, KPT)])


@jax.jit
def _run(ego_flat, nb_flat):
    mesh = plsc.VectorSubcoreMesh(core_axis_name="c", subcore_axis_name="s")
    f = pl.kernel(
        _sc_body,
        out_type=(jax.ShapeDtypeStruct((K,), jnp.int32),
                  jax.ShapeDtypeStruct((K,), jnp.int32)),
        mesh=mesh,
        compiler_params=pltpu.CompilerParams(needs_layout_passes=False),
        scratch_types=[
            pltpu.VMEM((C,), jnp.float32),
            pltpu.VMEM((C,), jnp.float32),
            pltpu.VMEM((C,), jnp.float32),
            pltpu.VMEM((C,), jnp.int32),
            pltpu.VMEM((HC,), jnp.int32),
            pltpu.VMEM((HC,), jnp.int32),
            pltpu.VMEM((HCOPIES * BINS,), jnp.int32),
            pltpu.VMEM((BINS,), jnp.int32),
            pltpu.VMEM((BINS,), jnp.int32),
            pltpu.VMEM((SLICE,), jnp.int32),
            pltpu.VMEM((SLICE,), jnp.int32),
            pltpu.VMEM((2 * T,), jnp.int32),
            pltpu.VMEM_SHARED((N,), jnp.float32),
            pltpu.VMEM_SHARED((N,), jnp.int32),
            pltpu.VMEM_SHARED((GRID,), jnp.int32),
            pltpu.VMEM_SHARED((2 * T,), jnp.int32),
            pltpu.SemaphoreType.DMA,
            pltpu.SemaphoreType.DMA,
            pltpu.SemaphoreType.DMA,
            pltpu.SemaphoreType.DMA,
        ],
    )
    return f(ego_flat, nb_flat)


def kernel(ego_conf, nb_conf, delta=0.25):
    del delta
    rows_idx, cols_idx = _run(ego_conf.reshape(-1), nb_conf.reshape(-1))
    return (rows_idx, cols_idx)
